# baseline (speedup 1.0000x reference)
import functools

import jax
import jax.numpy as jnp
from jax.experimental import pallas as pl
from jax.experimental.pallas import tpu as pltpu

N = 50000
ROW_BLK = 10000


def _bn_relu_body(y_ref, m_ref, s_ref, g_ref, be_ref, o_ref):
    y = y_ref[...]
    o_ref[...] = jnp.maximum((y - m_ref[...]) * s_ref[...] * g_ref[...] + be_ref[...], 0.0)


def _bn_relu(y, m, s, g, be):
    d = y.shape[1]
    return pl.pallas_call(
        _bn_relu_body,
        grid=(N // ROW_BLK,),
        in_specs=[
            pl.BlockSpec((ROW_BLK, d), lambda i: (i, 0)),
            pl.BlockSpec((1, d), lambda i: (0, 0)),
            pl.BlockSpec((1, d), lambda i: (0, 0)),
            pl.BlockSpec((1, d), lambda i: (0, 0)),
            pl.BlockSpec((1, d), lambda i: (0, 0)),
        ],
        out_specs=pl.BlockSpec((ROW_BLK, d), lambda i: (i, 0)),
        out_shape=jax.ShapeDtypeStruct((N, d), jnp.float32),
    )(y, m.reshape(1, d), s.reshape(1, d), g.reshape(1, d), be.reshape(1, d))


def kernel(x, edge_index, edge_weight,
           Wr0, Wn0, b0, g0, be0,
           Wr1, Wn1, b1, g1, be1,
           Wr2, Wn2, b2, g2, be2,
           Wr3, Wn3, b3, g3, be3,
           Wr4, Wn4, b4):
    src = edge_index[0]
    dst = edge_index[1]
    ew = edge_weight

    def spmm(p):
        msg = p[src] * ew[:, None]
        return jax.ops.segment_sum(msg, dst, num_segments=N)

    agg = spmm(x)
    y = x @ Wr0 + agg @ Wn0 + b0
    m = jnp.mean(y, axis=0)
    v = jnp.var(y, axis=0)
    h = _bn_relu(y, m, jax.lax.rsqrt(v + 1e-5), g0, be0)

    for Wr, Wn, b, g, be in ((Wr1, Wn1, b1, g1, be1),
                             (Wr2, Wn2, b2, g2, be2),
                             (Wr3, Wn3, b3, g3, be3)):
        p = h @ Wn
        agg = spmm(p)
        y = h @ Wr + agg + b
        m = jnp.mean(y, axis=0)
        v = jnp.var(y, axis=0)
        h = _bn_relu(y, m, jax.lax.rsqrt(v + 1e-5), g, be)

    p = h @ Wn4
    agg = spmm(p)
    out = h @ Wr4 + agg + b4
    return out.squeeze(1)


# speedup vs baseline: 1.4420x; 1.4420x over previous
import functools

import jax
import jax.numpy as jnp
from jax import lax
from jax.experimental import pallas as pl
from jax.experimental.pallas import tpu as pltpu
from jax.experimental.pallas import tpu_sc as plsc

N = 50000
E = 800000
D = 256
NC = 2
NS = 16
LANES = 16
GRP = 128
NGRP = E // GRP
SLAB = 32
NSLAB = D // SLAB
ROUNDS = NSLAB // NC
RPS = 3128
NP = RPS * NS

BLK = 2000
GRID = N // BLK


def _stage_group(src_hbm, dst_hbm, ew_hbm, base, srcb, dstb, ewb):
    pltpu.sync_copy(src_hbm.at[pl.ds(base, GRP)], srcb)
    pltpu.sync_copy(dst_hbm.at[pl.ds(base, GRP)], dstb)
    pltpu.sync_copy(ew_hbm.at[pl.ds(base, GRP)], ewb)


def _scale_rows(rows, ewb, width):
    def body(q, carry):
        ewv = ewb[pl.ds(q * LANES, LANES)]
        for l in range(LANES):
            j = q * LANES + l
            ew_s = jnp.full((LANES,), ewv[l])
            for f0 in range(0, width, LANES):
                sl = pl.ds(f0, LANES)
                rows[j, sl] = rows[j, sl] * ew_s
        return carry
    lax.fori_loop(0, GRP // LANES, body, 0)


def _spmm_wide(p8f, src, dst, ew, zeros32):
    mesh = plsc.VectorSubcoreMesh(core_axis_name="c", subcore_axis_name="s")

    @functools.partial(
        pl.kernel,
        out_type=jax.ShapeDtypeStruct((NSLAB * NP, SLAB), jnp.float32),
        mesh=mesh,
        compiler_params=pltpu.CompilerParams(use_tc_tiling_on_sc=False),
        scratch_types=[
            pltpu.VMEM_SHARED((NP, SLAB), jnp.float32),
            pltpu.VMEM((GRP,), jnp.int32),
            pltpu.VMEM((GRP,), jnp.int32),
            pltpu.VMEM((GRP,), jnp.int32),
            pltpu.VMEM((GRP,), jnp.float32),
            pltpu.VMEM((GRP, SLAB), jnp.float32),
            pltpu.SemaphoreType.DMA,
        ],
    )
    def k(p_hbm, src_hbm, dst_hbm, ew_hbm, z_hbm, agg_hbm,
          spmem, srcb, srcb2, dstb, ewb, rows, gsem):
        c = lax.axis_index("c")
        s = lax.axis_index("s")
        ts = jnp.where(s < NGRP % NS, NGRP // NS + 1, NGRP // NS)
        row0 = s * RPS
        for r in range(ROUNDS):
            slab = c * ROUNDS + r
            pltpu.sync_copy(z_hbm.at[pl.ds(row0, RPS)],
                            spmem.at[pl.ds(row0, RPS)])
            plsc.subcore_barrier()
            off = slab * NP

            def group(k_it, _):
                g = s + NS * k_it
                base = g * GRP
                _stage_group(src_hbm, dst_hbm, ew_hbm, base, srcb, dstb, ewb)
                for q in range(GRP // LANES):
                    sl = pl.ds(q * LANES, LANES)
                    srcb2[sl] = srcb[sl] + off
                pltpu.async_copy(p_hbm.at[srcb2], rows, gsem).wait()
                _scale_rows(rows, ewb, SLAB)
                pltpu.sync_copy(rows, spmem.at[dstb], add=True)
                return _
            lax.fori_loop(0, ts, group, 0)
            plsc.subcore_barrier()
            pltpu.sync_copy(spmem.at[pl.ds(row0, RPS)],
                            agg_hbm.at[pl.ds(off + row0, RPS)])
            plsc.subcore_barrier()

    return k(p8f, src, dst, ew, zeros32)


def _spmm1(p, src, dst, ew, zeros1):
    mesh = plsc.VectorSubcoreMesh(core_axis_name="c", subcore_axis_name="s")
    ngrp_c = NGRP // NC

    @functools.partial(
        pl.kernel,
        out_type=jax.ShapeDtypeStruct((NC * NP,), jnp.float32),
        mesh=mesh,
        compiler_params=pltpu.CompilerParams(use_tc_tiling_on_sc=False),
        scratch_types=[
            pltpu.VMEM_SHARED((NP,), jnp.float32),
            pltpu.VMEM((GRP,), jnp.int32),
            pltpu.VMEM((GRP,), jnp.int32),
            pltpu.VMEM((GRP,), jnp.float32),
            pltpu.VMEM((GRP,), jnp.float32),
            pltpu.SemaphoreType.DMA,
        ],
    )
    def k(p_hbm, src_hbm, dst_hbm, ew_hbm, z_hbm, agg_hbm,
          spmem, srcb, dstb, ewb, rows, gsem):
        c = lax.axis_index("c")
        s_ = lax.axis_index("s")
        ts = jnp.where(s_ < ngrp_c % NS, ngrp_c // NS + 1, ngrp_c // NS)
        row0 = s_ * RPS
        pltpu.sync_copy(z_hbm.at[pl.ds(row0, RPS)],
                        spmem.at[pl.ds(row0, RPS)])
        plsc.subcore_barrier()

        def group(k_it, carry):
            g = NC * (s_ + NS * k_it) + c
            base = g * GRP
            _stage_group(src_hbm, dst_hbm, ew_hbm, base, srcb, dstb, ewb)
            pltpu.async_copy(p_hbm.at[srcb], rows, gsem).wait()
            for q in range(GRP // LANES):
                sl = pl.ds(q * LANES, LANES)
                rows[sl] = rows[sl] * ewb[sl]
            pltpu.sync_copy(rows, spmem.at[dstb], add=True)
            return carry
        lax.fori_loop(0, ts, group, 0)
        plsc.subcore_barrier()
        pltpu.sync_copy(spmem.at[pl.ds(row0, RPS)],
                        agg_hbm.at[pl.ds(c * NP + row0, RPS)])

    return k(p, src, dst, ew, zeros1)


def _p0_body(x_ref, wr_ref, wn_ref, hr_ref, p8_ref):
    xx = x_ref[...]
    hr_ref[...] = jnp.dot(xx, wr_ref[...], preferred_element_type=jnp.float32,
                     precision=jax.lax.Precision.HIGHEST)
    pn = jnp.dot(xx, wn_ref[...], preferred_element_type=jnp.float32,
                     precision=jax.lax.Precision.HIGHEST)
    for k in range(NSLAB):
        p8_ref[k] = pn[:, k * SLAB:(k + 1) * SLAB]


def _proj0(x, Wr0, Wn0):
    return pl.pallas_call(
        _p0_body,
        grid=(GRID,),
        in_specs=[
            pl.BlockSpec((BLK, 4), lambda i: (i, 0)),
            pl.BlockSpec((4, D), lambda i: (0, 0)),
            pl.BlockSpec((4, D), lambda i: (0, 0)),
        ],
        out_specs=[
            pl.BlockSpec((BLK, D), lambda i: (i, 0)),
            pl.BlockSpec((NSLAB, BLK, SLAB), lambda i: (0, i, 0)),
        ],
        out_shape=[
            jax.ShapeDtypeStruct((N, D), jnp.float32),
            jax.ShapeDtypeStruct((NSLAB, NP, SLAB), jnp.float32),
        ],
    )(x, Wr0, Wn0)


def _a_body(hr_ref, agg_ref, b_ref, y_ref, ps_ref, pss_ref):
    agg = jnp.concatenate([agg_ref[k] for k in range(NSLAB)], axis=1)
    y = hr_ref[...] + agg + b_ref[...]
    y_ref[...] = y
    ps_ref[0] = jnp.sum(y, 0, keepdims=True)
    pss_ref[0] = jnp.sum(y * y, 0, keepdims=True)


def _assemble(hr, agg8, b):
    return pl.pallas_call(
        _a_body,
        grid=(GRID,),
        in_specs=[
            pl.BlockSpec((BLK, D), lambda i: (i, 0)),
            pl.BlockSpec((NSLAB, BLK, SLAB), lambda i: (0, i, 0)),
            pl.BlockSpec((1, D), lambda i: (0, 0)),
        ],
        out_specs=[
            pl.BlockSpec((BLK, D), lambda i: (i, 0)),
            pl.BlockSpec((1, 1, D), lambda i: (i, 0, 0)),
            pl.BlockSpec((1, 1, D), lambda i: (i, 0, 0)),
        ],
        out_shape=[
            jax.ShapeDtypeStruct((N, D), jnp.float32),
            jax.ShapeDtypeStruct((GRID, 1, D), jnp.float32),
            jax.ShapeDtypeStruct((GRID, 1, D), jnp.float32),
        ],
    )(hr, agg8, b.reshape(1, D))


def _b_body(nslab_o, slab_o, y_ref, m_ref, s_ref, g_ref, be_ref,
            wr_ref, wn_ref, hr_ref, p8_ref):
    z = jnp.maximum(
        (y_ref[...] - m_ref[...]) * s_ref[...] * g_ref[...] + be_ref[...], 0.0)
    hr_ref[...] = jnp.dot(z, wr_ref[...], preferred_element_type=jnp.float32,
                     precision=jax.lax.Precision.HIGHEST)
    pn = jnp.dot(z, wn_ref[...], preferred_element_type=jnp.float32,
                     precision=jax.lax.Precision.HIGHEST)
    for k in range(nslab_o):
        p8_ref[k] = pn[:, k * slab_o:(k + 1) * slab_o]


def _bnrelu_proj(y, m, sinv, g, be, Wr, Wn):
    dout = Wr.shape[1]
    nslab_o = NSLAB if dout == D else 1
    slab_o = dout // nslab_o
    return pl.pallas_call(
        functools.partial(_b_body, nslab_o, slab_o),
        grid=(GRID,),
        in_specs=[
            pl.BlockSpec((BLK, D), lambda i: (i, 0)),
            pl.BlockSpec((1, D), lambda i: (0, 0)),
            pl.BlockSpec((1, D), lambda i: (0, 0)),
            pl.BlockSpec((1, D), lambda i: (0, 0)),
            pl.BlockSpec((1, D), lambda i: (0, 0)),
            pl.BlockSpec((D, dout), lambda i: (0, 0)),
            pl.BlockSpec((D, dout), lambda i: (0, 0)),
        ],
        out_specs=[
            pl.BlockSpec((BLK, dout), lambda i: (i, 0)),
            pl.BlockSpec((nslab_o, BLK, slab_o), lambda i: (0, i, 0)),
        ],
        out_shape=[
            jax.ShapeDtypeStruct((N, dout), jnp.float32),
            jax.ShapeDtypeStruct((nslab_o, NP, slab_o), jnp.float32),
        ],
    )(y, m.reshape(1, D), sinv.reshape(1, D), g.reshape(1, D),
      be.reshape(1, D), Wr, Wn)


def _stats(ps, pss):
    m = jnp.sum(ps[:, 0], 0) / N
    v = jnp.sum(pss[:, 0], 0) / N - m * m
    return m, jax.lax.rsqrt(v + 1e-5)


def kernel(x, edge_index, edge_weight,
           Wr0, Wn0, b0, g0, be0,
           Wr1, Wn1, b1, g1, be1,
           Wr2, Wn2, b2, g2, be2,
           Wr3, Wn3, b3, g3, be3,
           Wr4, Wn4, b4):
    src = edge_index[0]
    dst = edge_index[1]
    ew = edge_weight
    z32 = jnp.zeros((NP, SLAB), jnp.float32)
    z1 = jnp.zeros((NP,), jnp.float32)

    hr, p8 = _proj0(x, Wr0, Wn0)
    for (b, g, be, Wrn, Wnn) in ((b0, g0, be0, Wr1, Wn1),
                                 (b1, g1, be1, Wr2, Wn2),
                                 (b2, g2, be2, Wr3, Wn3),
                                 (b3, g3, be3, Wr4, Wn4)):
        agg8f = _spmm_wide(p8.reshape(NSLAB * NP, SLAB), src, dst, ew, z32)
        y, ps, pss = _assemble(hr, agg8f.reshape(NSLAB, NP, SLAB), b)
        m, sinv = _stats(ps, pss)
        hr, p8 = _bnrelu_proj(y, m, sinv, g, be, Wrn, Wnn)

    agg1 = _spmm1(p8.reshape(NP), src, dst, ew, z1).reshape(NC, NP)
    out = hr[:, 0] + agg1[0, :N] + agg1[1, :N] + b4[0]
    return out
